# baseline (speedup 1.0000x reference)
import functools

import jax
import jax.numpy as jnp
from jax import lax
from jax.experimental import pallas as pl
from jax.experimental.pallas import tpu as pltpu
from jax.experimental.pallas import tpu_sc as plsc

N = 100000
E = 3200000
F = 16
NITER = 5

NC, NS = 2, 16
NW = NC * NS
CS = 64
KC = 8
KW = 1568
E_PAD = NW * KW * CS
N_PAD = 100096
RPT = N_PAD // NS
ZB = RPT // 16


NB = KW // KC
GS = KC * CS
G_MAIN = E // GS
G_TAIL = (E_PAD - E) // GS


def _seg_body(gather: bool, s_hbm, e_hbm, tail_hbm, out_hbm,
              acc, ibuf, rows, zbuf, isem, gsem, ssem):
    cid = lax.axis_index("c")
    sid = lax.axis_index("s")
    wid = cid * NS + sid
    gbase = wid * NB

    def _zero(i, _):
        zbuf[i] = jnp.zeros((F,), jnp.float32)
        return 0
    lax.fori_loop(0, ZB, _zero, 0)
    if not gather:
        def _ones(i, _):
            rows[0, 0, i] = jnp.ones((F,), jnp.float32)
            return 0
        lax.fori_loop(0, CS, _ones, 0)
    base = sid * RPT
    zd = [pltpu.make_async_copy(zbuf, acc.at[pl.ds(base + j * ZB, ZB)], gsem)
          for j in range(RPT // ZB)]
    for d in zd:
        d.start()
    for d in zd:
        d.wait()
    plsc.subcore_barrier()

    def _fire_idx(j):
        g = gbase + j
        r = g % 3

        @pl.when(g < G_MAIN)
        def _():
            if gather:
                pltpu.async_copy(e_hbm.at[0, g], ibuf.at[r, 0], isem)
            pltpu.async_copy(e_hbm.at[1, g], ibuf.at[r, 1], isem)

        @pl.when(g >= G_MAIN)
        def _():
            if gather:
                pltpu.async_copy(tail_hbm.at[0, g - G_MAIN],
                                 ibuf.at[r, 0], isem)
            pltpu.async_copy(tail_hbm.at[1, g - G_MAIN],
                             ibuf.at[r, 1], isem)

    def _wait_idx(j):
        r = (gbase + j) % 3
        if gather:
            pltpu.make_async_copy(e_hbm.at[0, 0], ibuf.at[r, 0], isem).wait()
        pltpu.make_async_copy(e_hbm.at[1, 0], ibuf.at[r, 1], isem).wait()

    def _gather_descs(j):
        g = gbase + j
        return [pltpu.make_async_copy(s_hbm.at[ibuf.at[g % 3, 0, b]],
                                      rows.at[g % 2, b], gsem)
                for b in range(KC)]

    def _fire_scatters(j):
        g = gbase + j
        for b in range(KC):
            src = rows.at[g % 2, b] if gather else rows.at[0, 0]
            pltpu.async_copy(src, acc.at[ibuf.at[g % 3, 1, b]], ssem,
                             add=True)

    def _wait_scatters(j):
        g = gbase + j
        for b in range(KC):
            src = rows.at[g % 2, b] if gather else rows.at[0, 0]
            pltpu.make_async_copy(src, acc.at[ibuf.at[g % 3, 1, b]],
                                  ssem).wait()

    _fire_idx(0)
    _wait_idx(0)
    _fire_idx(1)
    if gather:
        for d in _gather_descs(0):
            d.start()

    def _group(j, _):
        @pl.when(j >= 1)
        def _():
            _wait_scatters(j + 1)
        @pl.when(j + 1 < NB)
        def _():
            _wait_idx(j + 1)
        @pl.when(j + 2 < NB)
        def _():
            _fire_idx(j + 2)
        if gather:
            for d in _gather_descs(j):
                d.wait()
            @pl.when(j + 1 < NB)
            def _():
                for d in _gather_descs(j + 1):
                    d.start()
        _fire_scatters(j)
        return 0
    lax.fori_loop(0, NB, _group, 0)
    _wait_scatters(NB - 1)
    plsc.subcore_barrier()

    pltpu.sync_copy(acc.at[pl.ds(base, RPT)], out_hbm.at[cid, pl.ds(base, RPT)])


def _make_seg_kernel(gather: bool):
    mesh = plsc.VectorSubcoreMesh(core_axis_name="c", subcore_axis_name="s",
                                  num_cores=NC, num_subcores=NS)
    return pl.kernel(
        functools.partial(_seg_body, gather),
        out_type=jax.ShapeDtypeStruct((NC, N_PAD, F), jnp.float32),
        mesh=mesh,
        scratch_types=[
            pltpu.VMEM_SHARED((N_PAD, F), jnp.float32),
            pltpu.VMEM((3, 2, KC, CS), jnp.int32),
            pltpu.VMEM((2, KC, CS, F), jnp.float32)
            if gather else pltpu.VMEM((1, 1, CS, F), jnp.float32),
            pltpu.VMEM((ZB, F), jnp.float32),
            pltpu.SemaphoreType.DMA,
            pltpu.SemaphoreType.DMA,
            pltpu.SemaphoreType.DMA,
        ],
        compiler_params=pltpu.CompilerParams(use_tc_tiling_on_sc=False),
    )


_prop_kernel = _make_seg_kernel(True)
_deg_kernel = _make_seg_kernel(False)


UCH = 391
UPW = N_PAD // NW


def _rsqrt16(x):
    i = lax.bitcast_convert_type(x, jnp.int32)
    i = jnp.int32(0x5F3759DF) - lax.shift_right_logical(i, 1)
    y = lax.bitcast_convert_type(i, jnp.float32)
    for _ in range(3):
        y = y * (1.5 - 0.5 * x * y * y)
    return y


def _pointwise_pipeline(wid, in_descs, out_descs, row_fn, isem, osem):
    NCH = UPW // UCH

    def _run_rows(par):
        def _row(i, _):
            for k in range(2):
                row_fn(par, 2 * i + k)
            return 0
        lax.fori_loop(0, UCH // 2, _row, 0)
        if UCH % 2:
            row_fn(par, UCH - 1)

    for d in in_descs(0, 0):
        d.start()
    for c in range(NCH):
        p = c % 2
        if c >= 2:
            for d in out_descs(c - 2, p):
                d.wait()
        if c + 1 < NCH:
            for d in in_descs(c + 1, 1 - p):
                d.start()
        for d in in_descs(c, p):
            d.wait()
        _run_rows(p)
        for d in out_descs(c, p):
            d.start()
    for c in range(max(NCH - 2, 0), NCH):
        for d in out_descs(c, c % 2):
            d.wait()


def _dis_body(degp_hbm, h_hbm, dis_hbm, s0_hbm, b, isem, osem):
    wid = lax.axis_index("c") * NS + lax.axis_index("s")

    def in_descs(c, par):
        r0 = wid * UPW + c * UCH
        return [
            pltpu.make_async_copy(degp_hbm.at[0, pl.ds(r0, UCH)],
                                  b.at[par, 0], isem),
            pltpu.make_async_copy(degp_hbm.at[1, pl.ds(r0, UCH)],
                                  b.at[par, 1], isem),
            pltpu.make_async_copy(h_hbm.at[pl.ds(r0, UCH)],
                                  b.at[par, 2], isem),
        ]

    def out_descs(c, par):
        r0 = wid * UPW + c * UCH
        return [
            pltpu.make_async_copy(b.at[par, 4],
                                  dis_hbm.at[pl.ds(r0, UCH)], osem),
            pltpu.make_async_copy(b.at[par, 5],
                                  s0_hbm.at[pl.ds(r0, UCH)], osem),
        ]

    def row_fn(par, i):
        d = _rsqrt16(b[par, 0, i] + b[par, 1, i] + 1.0)
        b[par, 4, i] = d
        b[par, 5, i] = b[par, 2, i] * d

    _pointwise_pipeline(wid, in_descs, out_descs, row_fn, isem, osem)


def _upd_body(p_hbm, sp_hbm, dis_hbm, cur_hbm, s_hbm, b, isem, osem):
    wid = lax.axis_index("c") * NS + lax.axis_index("s")

    def in_descs(c, par):
        r0 = wid * UPW + c * UCH
        return [
            pltpu.make_async_copy(p_hbm.at[0, pl.ds(r0, UCH)],
                                  b.at[par, 0], isem),
            pltpu.make_async_copy(p_hbm.at[1, pl.ds(r0, UCH)],
                                  b.at[par, 1], isem),
            pltpu.make_async_copy(sp_hbm.at[pl.ds(r0, UCH)],
                                  b.at[par, 2], isem),
            pltpu.make_async_copy(dis_hbm.at[pl.ds(r0, UCH)],
                                  b.at[par, 3], isem),
        ]

    def out_descs(c, par):
        r0 = wid * UPW + c * UCH
        return [
            pltpu.make_async_copy(b.at[par, 4],
                                  cur_hbm.at[pl.ds(r0, UCH)], osem),
            pltpu.make_async_copy(b.at[par, 5],
                                  s_hbm.at[pl.ds(r0, UCH)], osem),
        ]

    def row_fn(par, i):
        cur = b[par, 3, i] * (b[par, 0, i] + b[par, 1, i] + b[par, 2, i])
        b[par, 4, i] = cur
        b[par, 5, i] = cur * b[par, 3, i]

    _pointwise_pipeline(wid, in_descs, out_descs, row_fn, isem, osem)


def _make_mesh():
    return plsc.VectorSubcoreMesh(core_axis_name="c", subcore_axis_name="s",
                                  num_cores=NC, num_subcores=NS)


_ptw_scratch = [
    pltpu.VMEM((2, 6, UCH, F), jnp.float32),
    pltpu.SemaphoreType.DMA,
    pltpu.SemaphoreType.DMA,
]

_dis_kernel = pl.kernel(
    _dis_body,
    out_type=(jax.ShapeDtypeStruct((N_PAD, F), jnp.float32),
              jax.ShapeDtypeStruct((N_PAD, F), jnp.float32)),
    mesh=_make_mesh(),
    scratch_types=_ptw_scratch,
    compiler_params=pltpu.CompilerParams(use_tc_tiling_on_sc=False),
)

_upd_kernel = pl.kernel(
    _upd_body,
    out_type=(jax.ShapeDtypeStruct((N_PAD, F), jnp.float32),
              jax.ShapeDtypeStruct((N_PAD, F), jnp.float32)),
    mesh=_make_mesh(),
    scratch_types=_ptw_scratch,
    compiler_params=pltpu.CompilerParams(use_tc_tiling_on_sc=False),
)


def _mlp_body(x_ref, w1_ref, b1_ref, w2_ref, b2_ref, o_ref):
    h = jnp.dot(x_ref[...], w1_ref[...], preferred_element_type=jnp.float32)
    h = jnp.maximum(h + b1_ref[...], 0.0)
    o_ref[...] = jnp.dot(h, w2_ref[...],
                         preferred_element_type=jnp.float32) + b2_ref[...]


def _mlp(x, W1, b1, W2, b2):
    BLK = 3128
    return pl.pallas_call(
        _mlp_body,
        grid=(N_PAD // BLK,),
        in_specs=[
            pl.BlockSpec((BLK, 128), lambda i: (i, 0)),
            pl.BlockSpec((128, 64), lambda i: (0, 0)),
            pl.BlockSpec((1, 64), lambda i: (0, 0)),
            pl.BlockSpec((64, F), lambda i: (0, 0)),
            pl.BlockSpec((1, F), lambda i: (0, 0)),
        ],
        out_specs=pl.BlockSpec((BLK, F), lambda i: (i, 0)),
        out_shape=jax.ShapeDtypeStruct((N_PAD, F), jnp.float32),
    )(x, W1, b1.reshape(1, 64), W2, b2.reshape(1, F))


def kernel(x, edge_index, W1, b1, W2, b2, Wh, bh):
    h = _mlp(x, W1, b1, W2, b2)

    e_hbm = edge_index.reshape(2, G_MAIN, KC, CS)
    ar = jnp.arange(E_PAD - E, dtype=jnp.int32)
    tail = jnp.stack([ar % N, N + ar % (N_PAD - N)]
                     ).reshape(2, G_TAIL, KC, CS)

    degp = _deg_kernel(h, e_hbm, tail)
    dis16, s = _dis_kernel(degp, h)

    preds = [h[:N]]
    hls = []
    for i in range(NITER):
        part = _prop_kernel(s, e_hbm, tail)
        cur, s = _upd_kernel(part, s, dis16)
        preds.append(cur[:N])
        hls.append(cur[:N] @ Wh + bh)

    lam = jax.nn.sigmoid(jnp.concatenate(hls, axis=1))
    p_list = []
    remaining = jnp.ones((N, 1), jnp.float32)
    for i in range(NITER):
        p_n = lam[:, i:i + 1] * remaining
        p_list.append(p_n)
        remaining = remaining * (1.0 - lam[:, i:i + 1])
    p_list[-1] = p_list[-1] + remaining
    p = jnp.concatenate(p_list, axis=1)
    return (jnp.stack(preds, axis=0), p)


# speedup vs baseline: 1.0016x; 1.0016x over previous
import functools

import jax
import jax.numpy as jnp
from jax import lax
from jax.experimental import pallas as pl
from jax.experimental.pallas import tpu as pltpu
from jax.experimental.pallas import tpu_sc as plsc

N = 100000
E = 3200000
F = 16
NITER = 5

NC, NS = 2, 16
NW = NC * NS
CS = 128
KC = 4
KW = 784
E_PAD = NW * KW * CS
N_PAD = 100096
RPT = N_PAD // NS
ZB = RPT // 16


NB = KW // KC
GS = KC * CS
G_MAIN = E // GS
G_TAIL = (E_PAD - E) // GS


def _seg_body(gather: bool, s_hbm, e_hbm, tail_hbm, out_hbm,
              acc, ibuf, rows, zbuf, isem, gsem, ssem):
    cid = lax.axis_index("c")
    sid = lax.axis_index("s")
    wid = cid * NS + sid
    gbase = wid * NB

    def _zero(i, _):
        zbuf[i] = jnp.zeros((F,), jnp.float32)
        return 0
    lax.fori_loop(0, ZB, _zero, 0)
    if not gather:
        def _ones(i, _):
            rows[0, 0, i] = jnp.ones((F,), jnp.float32)
            return 0
        lax.fori_loop(0, CS, _ones, 0)
    base = sid * RPT
    zd = [pltpu.make_async_copy(zbuf, acc.at[pl.ds(base + j * ZB, ZB)], gsem)
          for j in range(RPT // ZB)]
    for d in zd:
        d.start()
    for d in zd:
        d.wait()
    plsc.subcore_barrier()

    def _fire_idx(j):
        g = gbase + j
        r = g % 3

        @pl.when(g < G_MAIN)
        def _():
            if gather:
                pltpu.async_copy(e_hbm.at[0, g], ibuf.at[r, 0], isem)
            pltpu.async_copy(e_hbm.at[1, g], ibuf.at[r, 1], isem)

        @pl.when(g >= G_MAIN)
        def _():
            if gather:
                pltpu.async_copy(tail_hbm.at[0, g - G_MAIN],
                                 ibuf.at[r, 0], isem)
            pltpu.async_copy(tail_hbm.at[1, g - G_MAIN],
                             ibuf.at[r, 1], isem)

    def _wait_idx(j):
        r = (gbase + j) % 3
        if gather:
            pltpu.make_async_copy(e_hbm.at[0, 0], ibuf.at[r, 0], isem).wait()
        pltpu.make_async_copy(e_hbm.at[1, 0], ibuf.at[r, 1], isem).wait()

    def _gather_descs(j):
        g = gbase + j
        return [pltpu.make_async_copy(s_hbm.at[ibuf.at[g % 3, 0, b]],
                                      rows.at[g % 2, b], gsem)
                for b in range(KC)]

    def _fire_scatters(j):
        g = gbase + j
        for b in range(KC):
            src = rows.at[g % 2, b] if gather else rows.at[0, 0]
            pltpu.async_copy(src, acc.at[ibuf.at[g % 3, 1, b]], ssem,
                             add=True)

    def _wait_scatters(j):
        g = gbase + j
        for b in range(KC):
            src = rows.at[g % 2, b] if gather else rows.at[0, 0]
            pltpu.make_async_copy(src, acc.at[ibuf.at[g % 3, 1, b]],
                                  ssem).wait()

    _fire_idx(0)
    _wait_idx(0)
    _fire_idx(1)
    if gather:
        for d in _gather_descs(0):
            d.start()

    def _group(j, _):
        @pl.when(j >= 1)
        def _():
            _wait_scatters(j + 1)
        @pl.when(j + 1 < NB)
        def _():
            _wait_idx(j + 1)
        @pl.when(j + 2 < NB)
        def _():
            _fire_idx(j + 2)
        if gather:
            for d in _gather_descs(j):
                d.wait()
            @pl.when(j + 1 < NB)
            def _():
                for d in _gather_descs(j + 1):
                    d.start()
        _fire_scatters(j)
        return 0
    lax.fori_loop(0, NB, _group, 0)
    _wait_scatters(NB - 1)
    plsc.subcore_barrier()

    pltpu.sync_copy(acc.at[pl.ds(base, RPT)], out_hbm.at[cid, pl.ds(base, RPT)])


def _make_seg_kernel(gather: bool):
    mesh = plsc.VectorSubcoreMesh(core_axis_name="c", subcore_axis_name="s",
                                  num_cores=NC, num_subcores=NS)
    return pl.kernel(
        functools.partial(_seg_body, gather),
        out_type=jax.ShapeDtypeStruct((NC, N_PAD, F), jnp.float32),
        mesh=mesh,
        scratch_types=[
            pltpu.VMEM_SHARED((N_PAD, F), jnp.float32),
            pltpu.VMEM((3, 2, KC, CS), jnp.int32),
            pltpu.VMEM((2, KC, CS, F), jnp.float32)
            if gather else pltpu.VMEM((1, 1, CS, F), jnp.float32),
            pltpu.VMEM((ZB, F), jnp.float32),
            pltpu.SemaphoreType.DMA,
            pltpu.SemaphoreType.DMA,
            pltpu.SemaphoreType.DMA,
        ],
        compiler_params=pltpu.CompilerParams(use_tc_tiling_on_sc=False),
    )


_prop_kernel = _make_seg_kernel(True)
_deg_kernel = _make_seg_kernel(False)


UCH = 391
UPW = N_PAD // NW


def _rsqrt16(x):
    i = lax.bitcast_convert_type(x, jnp.int32)
    i = jnp.int32(0x5F3759DF) - lax.shift_right_logical(i, 1)
    y = lax.bitcast_convert_type(i, jnp.float32)
    for _ in range(3):
        y = y * (1.5 - 0.5 * x * y * y)
    return y


def _pointwise_pipeline(wid, in_descs, out_descs, row_fn, isem, osem):
    NCH = UPW // UCH

    def _run_rows(par):
        def _row(i, _):
            for k in range(2):
                row_fn(par, 2 * i + k)
            return 0
        lax.fori_loop(0, UCH // 2, _row, 0)
        if UCH % 2:
            row_fn(par, UCH - 1)

    for d in in_descs(0, 0):
        d.start()
    for c in range(NCH):
        p = c % 2
        if c >= 2:
            for d in out_descs(c - 2, p):
                d.wait()
        if c + 1 < NCH:
            for d in in_descs(c + 1, 1 - p):
                d.start()
        for d in in_descs(c, p):
            d.wait()
        _run_rows(p)
        for d in out_descs(c, p):
            d.start()
    for c in range(max(NCH - 2, 0), NCH):
        for d in out_descs(c, c % 2):
            d.wait()


def _dis_body(degp_hbm, h_hbm, dis_hbm, s0_hbm, b, isem, osem):
    wid = lax.axis_index("c") * NS + lax.axis_index("s")

    def in_descs(c, par):
        r0 = wid * UPW + c * UCH
        return [
            pltpu.make_async_copy(degp_hbm.at[0, pl.ds(r0, UCH)],
                                  b.at[par, 0], isem),
            pltpu.make_async_copy(degp_hbm.at[1, pl.ds(r0, UCH)],
                                  b.at[par, 1], isem),
            pltpu.make_async_copy(h_hbm.at[pl.ds(r0, UCH)],
                                  b.at[par, 2], isem),
        ]

    def out_descs(c, par):
        r0 = wid * UPW + c * UCH
        return [
            pltpu.make_async_copy(b.at[par, 4],
                                  dis_hbm.at[pl.ds(r0, UCH)], osem),
            pltpu.make_async_copy(b.at[par, 5],
                                  s0_hbm.at[pl.ds(r0, UCH)], osem),
        ]

    def row_fn(par, i):
        d = _rsqrt16(b[par, 0, i] + b[par, 1, i] + 1.0)
        b[par, 4, i] = d
        b[par, 5, i] = b[par, 2, i] * d

    _pointwise_pipeline(wid, in_descs, out_descs, row_fn, isem, osem)


def _upd_body(p_hbm, sp_hbm, dis_hbm, cur_hbm, s_hbm, b, isem, osem):
    wid = lax.axis_index("c") * NS + lax.axis_index("s")

    def in_descs(c, par):
        r0 = wid * UPW + c * UCH
        return [
            pltpu.make_async_copy(p_hbm.at[0, pl.ds(r0, UCH)],
                                  b.at[par, 0], isem),
            pltpu.make_async_copy(p_hbm.at[1, pl.ds(r0, UCH)],
                                  b.at[par, 1], isem),
            pltpu.make_async_copy(sp_hbm.at[pl.ds(r0, UCH)],
                                  b.at[par, 2], isem),
            pltpu.make_async_copy(dis_hbm.at[pl.ds(r0, UCH)],
                                  b.at[par, 3], isem),
        ]

    def out_descs(c, par):
        r0 = wid * UPW + c * UCH
        return [
            pltpu.make_async_copy(b.at[par, 4],
                                  cur_hbm.at[pl.ds(r0, UCH)], osem),
            pltpu.make_async_copy(b.at[par, 5],
                                  s_hbm.at[pl.ds(r0, UCH)], osem),
        ]

    def row_fn(par, i):
        cur = b[par, 3, i] * (b[par, 0, i] + b[par, 1, i] + b[par, 2, i])
        b[par, 4, i] = cur
        b[par, 5, i] = cur * b[par, 3, i]

    _pointwise_pipeline(wid, in_descs, out_descs, row_fn, isem, osem)


def _make_mesh():
    return plsc.VectorSubcoreMesh(core_axis_name="c", subcore_axis_name="s",
                                  num_cores=NC, num_subcores=NS)


_ptw_scratch = [
    pltpu.VMEM((2, 6, UCH, F), jnp.float32),
    pltpu.SemaphoreType.DMA,
    pltpu.SemaphoreType.DMA,
]

_dis_kernel = pl.kernel(
    _dis_body,
    out_type=(jax.ShapeDtypeStruct((N_PAD, F), jnp.float32),
              jax.ShapeDtypeStruct((N_PAD, F), jnp.float32)),
    mesh=_make_mesh(),
    scratch_types=_ptw_scratch,
    compiler_params=pltpu.CompilerParams(use_tc_tiling_on_sc=False),
)

_upd_kernel = pl.kernel(
    _upd_body,
    out_type=(jax.ShapeDtypeStruct((N_PAD, F), jnp.float32),
              jax.ShapeDtypeStruct((N_PAD, F), jnp.float32)),
    mesh=_make_mesh(),
    scratch_types=_ptw_scratch,
    compiler_params=pltpu.CompilerParams(use_tc_tiling_on_sc=False),
)


def _mlp_body(x_ref, w1_ref, b1_ref, w2_ref, b2_ref, o_ref):
    h = jnp.dot(x_ref[...], w1_ref[...], preferred_element_type=jnp.float32)
    h = jnp.maximum(h + b1_ref[...], 0.0)
    o_ref[...] = jnp.dot(h, w2_ref[...],
                         preferred_element_type=jnp.float32) + b2_ref[...]


def _mlp(x, W1, b1, W2, b2):
    BLK = 3128
    return pl.pallas_call(
        _mlp_body,
        grid=(N_PAD // BLK,),
        in_specs=[
            pl.BlockSpec((BLK, 128), lambda i: (i, 0)),
            pl.BlockSpec((128, 64), lambda i: (0, 0)),
            pl.BlockSpec((1, 64), lambda i: (0, 0)),
            pl.BlockSpec((64, F), lambda i: (0, 0)),
            pl.BlockSpec((1, F), lambda i: (0, 0)),
        ],
        out_specs=pl.BlockSpec((BLK, F), lambda i: (i, 0)),
        out_shape=jax.ShapeDtypeStruct((N_PAD, F), jnp.float32),
    )(x, W1, b1.reshape(1, 64), W2, b2.reshape(1, F))


def kernel(x, edge_index, W1, b1, W2, b2, Wh, bh):
    h = _mlp(x, W1, b1, W2, b2)

    e_hbm = edge_index.reshape(2, G_MAIN, KC, CS)
    ar = jnp.arange(E_PAD - E, dtype=jnp.int32)
    tail = jnp.stack([ar % N, N + ar % (N_PAD - N)]
                     ).reshape(2, G_TAIL, KC, CS)

    degp = _deg_kernel(h, e_hbm, tail)
    dis16, s = _dis_kernel(degp, h)

    preds = [h[:N]]
    hls = []
    for i in range(NITER):
        part = _prop_kernel(s, e_hbm, tail)
        cur, s = _upd_kernel(part, s, dis16)
        preds.append(cur[:N])
        hls.append(cur[:N] @ Wh + bh)

    lam = jax.nn.sigmoid(jnp.concatenate(hls, axis=1))
    p_list = []
    remaining = jnp.ones((N, 1), jnp.float32)
    for i in range(NITER):
        p_n = lam[:, i:i + 1] * remaining
        p_list.append(p_n)
        remaining = remaining * (1.0 - lam[:, i:i + 1])
    p_list[-1] = p_list[-1] + remaining
    p = jnp.concatenate(p_list, axis=1)
    return (jnp.stack(preds, axis=0), p)
